# Initial kernel scaffold; baseline (speedup 1.0000x reference)
import jax
import jax.numpy as jnp
from jax.experimental import pallas as pl


def kernel(x, position_embedding):
    raise NotImplementedError("write your pallas kernel here")


# speedup vs baseline: 1.5541x; 1.5541x over previous
import functools

import jax
import jax.numpy as jnp
from jax import lax
from jax.experimental import pallas as pl
from jax.experimental.pallas import tpu as pltpu
from jax.experimental.pallas import tpu_sc as plsc


def _make_bcast_kernel(batch, seq, dim):
    info = plsc.get_sparse_core_info()
    nc, ns = info.num_cores, info.num_subcores
    nw = nc * ns
    assert seq % nw == 0
    rows_per_w = seq // nw
    chunk = 16
    while rows_per_w % chunk:
        chunk //= 2
    n_chunks = rows_per_w // chunk

    mesh = plsc.VectorSubcoreMesh(core_axis_name="c", subcore_axis_name="s")

    @functools.partial(
        pl.kernel,
        mesh=mesh,
        out_type=jax.ShapeDtypeStruct((batch, seq, dim), jnp.float32),
        scratch_types=[
            pltpu.VMEM((chunk, dim), jnp.float32),
        ],
    )
    def bcast(table_hbm, out_hbm, buf):
        wid = lax.axis_index("s") * nc + lax.axis_index("c")
        base = wid * rows_per_w
        for i in range(n_chunks):
            row0 = base + i * chunk
            pltpu.sync_copy(table_hbm.at[pl.ds(row0, chunk)], buf)
            for b in range(batch):
                pltpu.sync_copy(buf, out_hbm.at[b, pl.ds(row0, chunk)])

    return bcast


def kernel(x, position_embedding):
    batch, seq, dim = x.shape
    fn = _make_bcast_kernel(batch, seq, dim)
    return fn(position_embedding)


# speedup vs baseline: 1.6224x; 1.0439x over previous
import functools

import jax
import jax.numpy as jnp
from jax import lax
from jax.experimental import pallas as pl
from jax.experimental.pallas import tpu as pltpu
from jax.experimental.pallas import tpu_sc as plsc


def _make_bcast_kernel(batch, seq, dim):
    info = plsc.get_sparse_core_info()
    nc, ns = info.num_cores, info.num_subcores
    nw = nc * ns
    assert seq % nw == 0
    rows_per_w = seq // nw
    chunk = 16
    while rows_per_w % chunk:
        chunk //= 2
    n_chunks = rows_per_w // chunk

    mesh = plsc.VectorSubcoreMesh(core_axis_name="c", subcore_axis_name="s")

    @functools.partial(
        pl.kernel,
        mesh=mesh,
        out_type=jax.ShapeDtypeStruct((batch, seq, dim), jnp.float32),
        scratch_types=[
            pltpu.VMEM((chunk, dim), jnp.float32),
            pltpu.VMEM((chunk, dim), jnp.float32),
            pltpu.SemaphoreType.DMA,
            pltpu.SemaphoreType.DMA,
        ],
    )
    def bcast(table_hbm, out_hbm, buf0, buf1, rsem, wsem):
        wid = lax.axis_index("s") * nc + lax.axis_index("c")
        base = wid * rows_per_w
        bufs = (buf0, buf1)
        reads = [None, None]
        writes = [None, None]
        reads[0] = pltpu.async_copy(
            table_hbm.at[pl.ds(base, chunk)], bufs[0], rsem
        )
        for i in range(n_chunks):
            cur = i % 2
            nxt = (i + 1) % 2
            if i + 1 < n_chunks:
                if writes[nxt] is not None:
                    for d in writes[nxt]:
                        d.wait()
                    writes[nxt] = None
                row_n = base + (i + 1) * chunk
                reads[nxt] = pltpu.async_copy(
                    table_hbm.at[pl.ds(row_n, chunk)], bufs[nxt], rsem
                )
            reads[cur].wait()
            row0 = base + i * chunk
            writes[cur] = [
                pltpu.async_copy(
                    bufs[cur], out_hbm.at[b, pl.ds(row0, chunk)], wsem
                )
                for b in range(batch)
            ]
        for pending in writes:
            if pending is not None:
                for d in pending:
                    d.wait()

    return bcast


def kernel(x, position_embedding):
    batch, seq, dim = x.shape
    fn = _make_bcast_kernel(batch, seq, dim)
    return fn(position_embedding)
